# Initial kernel scaffold; baseline (speedup 1.0000x reference)
import jax
import jax.numpy as jnp
from jax.experimental import pallas as pl


def kernel(X, shuffled_idx):
    raise NotImplementedError("write your pallas kernel here")


# speedup vs baseline: 3.4667x; 3.4667x over previous
import functools

import jax
import jax.numpy as jnp
from jax import lax
from jax.experimental import pallas as pl
from jax.experimental.pallas import tpu as pltpu
from jax.experimental.pallas import tpu_sc as plsc

B, C, E, P = 32, 16, 128, 512
NUM_PERM = 1000

NC, NS, L = 2, 16, 16
NW = NC * NS
PAIRS_PER_W = (B * C) // NW
CE = 32
NCH = E // CE
CHW = CE * P


def _shuffle_gather(x_flat, order):
    mesh = plsc.VectorSubcoreMesh(core_axis_name="c", subcore_axis_name="s")

    @functools.partial(
        pl.kernel,
        out_type=jax.ShapeDtypeStruct((B * C * E * P,), jnp.float32),
        mesh=mesh,
        compiler_params=pltpu.CompilerParams(needs_layout_passes=False),
        scratch_types=[
            pltpu.VMEM((P,), jnp.int32),
            pltpu.VMEM((CHW,), jnp.float32),
            pltpu.VMEM((CHW,), jnp.float32),
        ],
    )
    def k(x_hbm, idx_hbm, out_hbm, idxbuf, inbuf, outbuf):
        cid = lax.axis_index("c")
        sid = lax.axis_index("s")
        w = sid * NC + cid
        c = w // 2
        bh = w % 2
        pltpu.sync_copy(idx_hbm.at[c], idxbuf)

        def pair_body(kk, _):
            b = bh * (B // 2) + kk
            pair = b * C + c

            def chunk_body(ch, _):
                base = pair * (E * P) + ch * CHW
                pltpu.sync_copy(x_hbm.at[pl.ds(base, CHW)], inbuf)

                def j_body(j, _):
                    idxv = idxbuf[pl.ds(j * L, L)]

                    def e_body(e, iv):
                        vals = plsc.load_gather(inbuf, [iv])
                        outbuf[pl.ds(e * P + j * L, L)] = vals
                        return iv + P

                    lax.fori_loop(0, CE, e_body, idxv, unroll=4)
                    return 0

                lax.fori_loop(0, P // L, j_body, 0)
                pltpu.sync_copy(outbuf, out_hbm.at[pl.ds(base, CHW)])
                return 0

            lax.fori_loop(0, NCH, chunk_body, 0)
            return 0

        lax.fori_loop(0, PAIRS_PER_W, pair_body, 0)

    return k(x_flat, order)


def kernel(X, shuffled_idx):
    rand_idx = jax.random.randint(jax.random.key(1), (1,), 0, NUM_PERM - 1)[0]
    order = lax.dynamic_index_in_dim(
        shuffled_idx, rand_idx, axis=0, keepdims=False
    ).astype(jnp.int32)
    x_flat = X.reshape(B * C * E * P)
    out = _shuffle_gather(x_flat, order)
    return out.reshape(B, C, E, P)


# speedup vs baseline: 4.2129x; 1.2152x over previous
import functools

import jax
import jax.numpy as jnp
from jax import lax
from jax.experimental import pallas as pl
from jax.experimental.pallas import tpu as pltpu
from jax.experimental.pallas import tpu_sc as plsc

B, C, E, P = 32, 16, 128, 512
NUM_PERM = 1000

NC, NS, L = 2, 16, 16
NW = NC * NS
PAIRS_PER_W = (B * C) // NW
CE = 32
NCH = E // CE
CHW = CE * P
NU = PAIRS_PER_W * NCH


def _shuffle_gather(x_flat, order):
    mesh = plsc.VectorSubcoreMesh(core_axis_name="c", subcore_axis_name="s")

    @functools.partial(
        pl.kernel,
        out_type=jax.ShapeDtypeStruct((B * C * E * P,), jnp.float32),
        mesh=mesh,
        compiler_params=pltpu.CompilerParams(needs_layout_passes=False),
        scratch_types=[
            pltpu.VMEM((P,), jnp.int32),
            pltpu.VMEM((CHW,), jnp.float32),
            pltpu.VMEM((CHW,), jnp.float32),
            pltpu.VMEM((CHW,), jnp.float32),
            pltpu.VMEM((CHW,), jnp.float32),
            pltpu.SemaphoreType.DMA,
            pltpu.SemaphoreType.DMA,
            pltpu.SemaphoreType.DMA,
            pltpu.SemaphoreType.DMA,
        ],
    )
    def k(x_hbm, idx_hbm, out_hbm, idxbuf, in0, in1, out0, out1,
          isem0, isem1, osem0, osem1):
        cid = lax.axis_index("c")
        sid = lax.axis_index("s")
        w = sid * NC + cid
        c = w // 2
        bh = w % 2
        inb = (in0, in1)
        outb = (out0, out1)
        isem = (isem0, isem1)
        osem = (osem0, osem1)

        pltpu.sync_copy(idx_hbm.at[c], idxbuf)

        def unit_base(u):
            kk = u // NCH
            ch = u % NCH
            b = bh * (B // 2) + kk
            return (b * C + c) * (E * P) + ch * CHW

        def in_copy(u, par):
            return pltpu.make_async_copy(
                x_hbm.at[pl.ds(unit_base(u), CHW)], inb[par], isem[par])

        def out_copy(u, par):
            return pltpu.make_async_copy(
                outb[par], out_hbm.at[pl.ds(unit_base(u), CHW)], osem[par])

        def compute(inbuf, outbuf):
            def j_body(j, _):
                iv = idxbuf[pl.ds(j * L, L)]
                off = j * L
                for e in range(CE):
                    outbuf[pl.ds(off + e * P, L)] = plsc.load_gather(
                        inbuf, [iv + (e * P)])
                return 0

            lax.fori_loop(0, P // L, j_body, 0)

        in_copy(0, 0).start()

        def outer(i, _):
            for par in range(2):
                u = i * 2 + par

                @pl.when(u + 1 < NU)
                def _start_next():
                    in_copy(u + 1, 1 - par).start()

                in_copy(u, par).wait()

                @pl.when(u >= 2)
                def _drain_prev():
                    out_copy(u - 2, par).wait()

                compute(inb[par], outb[par])
                out_copy(u, par).start()
            return 0

        lax.fori_loop(0, NU // 2, outer, 0)
        out_copy(NU - 2, 0).wait()
        out_copy(NU - 1, 1).wait()

    return k(x_flat, order)


def kernel(X, shuffled_idx):
    rand_idx = jax.random.randint(jax.random.key(1), (1,), 0, NUM_PERM - 1)[0]
    order = lax.dynamic_index_in_dim(
        shuffled_idx, rand_idx, axis=0, keepdims=False
    ).astype(jnp.int32)
    x_flat = X.reshape(B * C * E * P)
    out = _shuffle_gather(x_flat, order)
    return out.reshape(B, C, E, P)


# speedup vs baseline: 6.8933x; 1.6362x over previous
import functools

import jax
import jax.numpy as jnp
from jax import lax
from jax.experimental import pallas as pl
from jax.experimental.pallas import tpu as pltpu
from jax.experimental.pallas import tpu_sc as plsc

B, C, E, P = 32, 16, 128, 512
NUM_PERM = 1000

NC, NS, L = 2, 16, 16
NW = NC * NS
PAIRS_PER_W = (B * C) // NW
CE = 32
NCH = E // CE
NU = PAIRS_PER_W * NCH


def _shuffle_gather(x3, order):
    mesh = plsc.VectorSubcoreMesh(core_axis_name="c", subcore_axis_name="s")

    @functools.partial(
        pl.kernel,
        out_type=jax.ShapeDtypeStruct((B * C, E, P), jnp.float32),
        mesh=mesh,
        compiler_params=pltpu.CompilerParams(needs_layout_passes=False),
        scratch_types=[
            pltpu.VMEM((P,), jnp.int32),
            pltpu.VMEM((CE, P), jnp.float32),
            pltpu.VMEM((CE, P), jnp.float32),
            pltpu.VMEM((CE, P), jnp.float32),
            pltpu.VMEM((CE, P), jnp.float32),
            pltpu.SemaphoreType.DMA,
            pltpu.SemaphoreType.DMA,
            pltpu.SemaphoreType.DMA,
            pltpu.SemaphoreType.DMA,
        ],
    )
    def k(x_hbm, idx_hbm, out_hbm, idxbuf, in0, in1, out0, out1,
          isem0, isem1, osem0, osem1):
        cid = lax.axis_index("c")
        sid = lax.axis_index("s")
        w = sid * NC + cid
        c = w // 2
        bh = w % 2
        inb = (in0, in1)
        outb = (out0, out1)
        isem = (isem0, isem1)
        osem = (osem0, osem1)

        pltpu.sync_copy(idx_hbm.at[c], idxbuf)

        def unit_slot(u):
            kk = u // NCH
            ch = u % NCH
            b = bh * (B // 2) + kk
            return b * C + c, ch * CE

        def in_copy(u, par):
            pair, e0 = unit_slot(u)
            return pltpu.make_async_copy(
                x_hbm.at[pair, pl.ds(e0, CE), :], inb[par], isem[par])

        def out_copy(u, par):
            pair, e0 = unit_slot(u)
            return pltpu.make_async_copy(
                outb[par], out_hbm.at[pair, pl.ds(e0, CE), :], osem[par])

        def compute(inbuf, outbuf):
            def j_body(j, _):
                iv = idxbuf[pl.ds(j * L, L)]
                for e in range(CE):
                    ev = jnp.full((L,), e, dtype=jnp.int32)
                    outbuf[e, pl.ds(j * L, L)] = plsc.load_gather(
                        inbuf, [ev, iv])
                return 0

            lax.fori_loop(0, P // L, j_body, 0)

        in_copy(0, 0).start()

        def outer(i, _):
            for par in range(2):
                u = i * 2 + par

                @pl.when(u + 1 < NU)
                def _start_next():
                    in_copy(u + 1, 1 - par).start()

                in_copy(u, par).wait()

                @pl.when(u >= 2)
                def _drain_prev():
                    out_copy(u - 2, par).wait()

                compute(inb[par], outb[par])
                out_copy(u, par).start()
            return 0

        lax.fori_loop(0, NU // 2, outer, 0)
        out_copy(NU - 2, 0).wait()
        out_copy(NU - 1, 1).wait()

    return k(x3, order)


def kernel(X, shuffled_idx):
    rand_idx = jax.random.randint(jax.random.key(1), (1,), 0, NUM_PERM - 1)[0]
    order = lax.dynamic_index_in_dim(
        shuffled_idx, rand_idx, axis=0, keepdims=False
    ).astype(jnp.int32)
    x3 = X.reshape(B * C, E, P)
    out = _shuffle_gather(x3, order)
    return out.reshape(B, C, E, P)


# speedup vs baseline: 13.5700x; 1.9686x over previous
import functools

import jax
import jax.numpy as jnp
from jax import lax
from jax.experimental import pallas as pl
from jax.experimental.pallas import tpu as pltpu
from jax.experimental.pallas import tpu_sc as plsc

B, C, E, P = 32, 16, 128, 512
NUM_PERM = 1000

NC, NS, L = 2, 16, 16
NW = NC * NS
PAIRS_PER_W = (B * C) // NW
CE = 32
NCH = E // CE
NU = PAIRS_PER_W * NCH


def _shuffle_gather(x3, order):
    mesh = plsc.VectorSubcoreMesh(core_axis_name="c", subcore_axis_name="s")

    @functools.partial(
        pl.kernel,
        out_type=jax.ShapeDtypeStruct((B * C, E, P), jnp.float32),
        mesh=mesh,
        compiler_params=pltpu.CompilerParams(needs_layout_passes=False),
        scratch_types=[
            pltpu.VMEM((P,), jnp.int32),
            pltpu.VMEM((CE, P), jnp.float32),
            pltpu.VMEM((CE, P), jnp.float32),
            pltpu.VMEM((CE, P), jnp.float32),
            pltpu.VMEM((CE, P), jnp.float32),
            pltpu.VMEM((CE, P), jnp.float32),
            pltpu.VMEM((CE, P), jnp.float32),
            pltpu.SemaphoreType.DMA,
            pltpu.SemaphoreType.DMA,
            pltpu.SemaphoreType.DMA,
            pltpu.SemaphoreType.DMA,
            pltpu.SemaphoreType.DMA,
            pltpu.SemaphoreType.DMA,
        ],
    )
    def k(x_hbm, idx_hbm, out_hbm, idxbuf, in0, in1, in2, in3, out0, out1,
          isem0, isem1, isem2, isem3, osem0, osem1):
        cid = lax.axis_index("c")
        sid = lax.axis_index("s")
        w = sid * NC + cid
        c = w // 2
        bh = w % 2
        inb = (in0, in1, in2, in3)
        outb = (out0, out1)
        isem = (isem0, isem1, isem2, isem3)
        osem = (osem0, osem1)

        pltpu.sync_copy(idx_hbm.at[c], idxbuf)

        def unit_slot(u):
            kk = u // NCH
            ch = u % NCH
            b = bh * (B // 2) + kk
            return b * C + c, ch * CE

        def in_copy(u, par):
            pair, e0 = unit_slot(u)
            return pltpu.make_async_copy(
                x_hbm.at[pair, pl.ds(e0, CE), :], inb[par], isem[par])

        def out_copy(u, par):
            pair, e0 = unit_slot(u)
            return pltpu.make_async_copy(
                outb[par], out_hbm.at[pair, pl.ds(e0, CE), :], osem[par])

        def compute(inbuf, outbuf):
            @plsc.parallel_loop(0, P // L)
            def j_body(j):
                iv = idxbuf[pl.ds(j * L, L)]

                @plsc.parallel_loop(0, CE, unroll=8)
                def e_body(e):
                    ev = jnp.full((L,), e, dtype=jnp.int32)
                    outbuf[e, pl.ds(j * L, L)] = plsc.load_gather(
                        inbuf, [ev, iv])

        in_copy(0, 0).start()
        in_copy(1, 1).start()
        in_copy(2, 2).start()

        def outer(i, _):
            for t in range(4):
                u = i * 4 + t
                op = t % 2

                @pl.when(u + 3 < NU)
                def _start_next():
                    in_copy(u + 3, (t + 3) % 4).start()

                in_copy(u, t).wait()

                @pl.when(u >= 2)
                def _drain_prev():
                    out_copy(u - 2, op).wait()

                compute(inb[t], outb[op])
                out_copy(u, op).start()
            return 0

        lax.fori_loop(0, NU // 4, outer, 0)
        out_copy(NU - 2, 0).wait()
        out_copy(NU - 1, 1).wait()

    return k(x3, order)


def kernel(X, shuffled_idx):
    rand_idx = jax.random.randint(jax.random.key(1), (1,), 0, NUM_PERM - 1)[0]
    order = lax.dynamic_index_in_dim(
        shuffled_idx, rand_idx, axis=0, keepdims=False
    ).astype(jnp.int32)
    x3 = X.reshape(B * C, E, P)
    out = _shuffle_gather(x3, order)
    return out.reshape(B, C, E, P)


# speedup vs baseline: 14.5786x; 1.0743x over previous
import functools

import jax
import jax.numpy as jnp
from jax import lax
from jax.experimental import pallas as pl
from jax.experimental.pallas import tpu as pltpu
from jax.experimental.pallas import tpu_sc as plsc

B, C, E, P = 32, 16, 128, 512
NUM_PERM = 1000

NC, NS, L = 2, 16, 16
NW = NC * NS
PAIRS_PER_W = (B * C) // NW
CE = 32
NCH = E // CE
NU = PAIRS_PER_W * NCH


def _shuffle_gather(x3, order):
    mesh = plsc.VectorSubcoreMesh(core_axis_name="c", subcore_axis_name="s")

    @functools.partial(
        pl.kernel,
        out_type=jax.ShapeDtypeStruct((B * C, E, P), jnp.float32),
        mesh=mesh,
        compiler_params=pltpu.CompilerParams(needs_layout_passes=False),
        scratch_types=[
            pltpu.VMEM((P,), jnp.int32),
            pltpu.VMEM((CE, P), jnp.float32),
            pltpu.VMEM((CE, P), jnp.float32),
            pltpu.VMEM((CE, P), jnp.float32),
            pltpu.VMEM((CE, P), jnp.float32),
            pltpu.SemaphoreType.DMA,
            pltpu.SemaphoreType.DMA,
            pltpu.SemaphoreType.DMA,
            pltpu.SemaphoreType.DMA,
        ],
    )
    def k(x_hbm, idx_hbm, out_hbm, idxbuf, in0, in1, out0, out1,
          isem0, isem1, osem0, osem1):
        cid = lax.axis_index("c")
        sid = lax.axis_index("s")
        w = sid * NC + cid
        c = w // 2
        bh = w % 2
        inb = (in0, in1)
        outb = (out0, out1)
        isem = (isem0, isem1)
        osem = (osem0, osem1)

        pltpu.sync_copy(idx_hbm.at[c], idxbuf)

        def unit_slot(u):
            kk = u // NCH
            ch = u % NCH
            b = bh * (B // 2) + kk
            return b * C + c, ch * CE

        def in_copy(u, par):
            pair, e0 = unit_slot(u)
            return pltpu.make_async_copy(
                x_hbm.at[pair, pl.ds(e0, CE), :], inb[par], isem[par])

        def out_copy(u, par):
            pair, e0 = unit_slot(u)
            return pltpu.make_async_copy(
                outb[par], out_hbm.at[pair, pl.ds(e0, CE), :], osem[par])

        def compute(inbuf, outbuf):
            @plsc.parallel_loop(0, P // L, unroll=2)
            def j_body(j):
                iv = idxbuf[pl.ds(j * L, L)]

                @plsc.parallel_loop(0, CE, unroll=8)
                def e_body(e):
                    ev = jnp.full((L,), e, dtype=jnp.int32)
                    outbuf[e, pl.ds(j * L, L)] = plsc.load_gather(
                        inbuf, [ev, iv])

        in_copy(0, 0).start()

        def outer(i, _):
            for par in range(2):
                u = i * 2 + par

                @pl.when(u + 1 < NU)
                def _start_next():
                    in_copy(u + 1, 1 - par).start()

                in_copy(u, par).wait()

                @pl.when(u >= 2)
                def _drain_prev():
                    out_copy(u - 2, par).wait()

                compute(inb[par], outb[par])
                out_copy(u, par).start()
            return 0

        lax.fori_loop(0, NU // 2, outer, 0)
        out_copy(NU - 2, 0).wait()
        out_copy(NU - 1, 1).wait()

    return k(x3, order)


def kernel(X, shuffled_idx):
    rand_idx = jax.random.randint(jax.random.key(1), (1,), 0, NUM_PERM - 1)[0]
    order = lax.dynamic_index_in_dim(
        shuffled_idx, rand_idx, axis=0, keepdims=False
    ).astype(jnp.int32)
    x3 = X.reshape(B * C, E, P)
    out = _shuffle_gather(x3, order)
    return out.reshape(B, C, E, P)


# speedup vs baseline: 15.1739x; 1.0408x over previous
import functools

import jax
import jax.numpy as jnp
from jax import lax
from jax.experimental import pallas as pl
from jax.experimental.pallas import tpu as pltpu
from jax.experimental.pallas import tpu_sc as plsc

B, C, E, P = 32, 16, 128, 512
NUM_PERM = 1000

NC, NS, L = 2, 16, 16
NW = NC * NS
PAIRS_PER_W = (B * C) // NW
CE = 32
NCH = E // CE
NU = PAIRS_PER_W * NCH


def _shuffle_gather(x3, order):
    mesh = plsc.VectorSubcoreMesh(core_axis_name="c", subcore_axis_name="s")

    @functools.partial(
        pl.kernel,
        out_type=jax.ShapeDtypeStruct((B * C, E, P), jnp.float32),
        mesh=mesh,
        compiler_params=pltpu.CompilerParams(needs_layout_passes=False),
        scratch_types=[
            pltpu.VMEM((P,), jnp.int32),
            pltpu.VMEM((CE, P), jnp.float32),
            pltpu.VMEM((CE, P), jnp.float32),
            pltpu.VMEM((CE, P), jnp.float32),
            pltpu.VMEM((CE, P), jnp.float32),
            pltpu.SemaphoreType.DMA,
            pltpu.SemaphoreType.DMA,
            pltpu.SemaphoreType.DMA,
            pltpu.SemaphoreType.DMA,
        ],
    )
    def k(x_hbm, idx_hbm, out_hbm, idxbuf, in0, in1, out0, out1,
          isem0, isem1, osem0, osem1):
        cid = lax.axis_index("c")
        sid = lax.axis_index("s")
        w = sid * NC + cid
        c = w // 2
        bh = w % 2
        inb = (in0, in1)
        outb = (out0, out1)
        isem = (isem0, isem1)
        osem = (osem0, osem1)

        pltpu.sync_copy(idx_hbm.at[c], idxbuf)

        def unit_slot(u):
            kk = u // NCH
            ch = u % NCH
            b = bh * (B // 2) + kk
            return b * C + c, ch * CE

        def in_copy(u, par):
            pair, e0 = unit_slot(u)
            return pltpu.make_async_copy(
                x_hbm.at[pair, pl.ds(e0, CE), :], inb[par], isem[par])

        def out_copy(u, par):
            pair, e0 = unit_slot(u)
            return pltpu.make_async_copy(
                outb[par], out_hbm.at[pair, pl.ds(e0, CE), :], osem[par])

        def compute(inbuf, outbuf):
            @plsc.parallel_loop(0, P // L, unroll=4)
            def j_body(j):
                iv = idxbuf[pl.ds(j * L, L)]

                @plsc.parallel_loop(0, CE, unroll=8)
                def e_body(e):
                    ev = jnp.full((L,), e, dtype=jnp.int32)
                    outbuf[e, pl.ds(j * L, L)] = plsc.load_gather(
                        inbuf, [ev, iv])

        in_copy(0, 0).start()

        def outer(i, _):
            for par in range(2):
                u = i * 2 + par

                @pl.when(u + 1 < NU)
                def _start_next():
                    in_copy(u + 1, 1 - par).start()

                in_copy(u, par).wait()

                @pl.when(u >= 2)
                def _drain_prev():
                    out_copy(u - 2, par).wait()

                compute(inb[par], outb[par])
                out_copy(u, par).start()
            return 0

        lax.fori_loop(0, NU // 2, outer, 0)
        out_copy(NU - 2, 0).wait()
        out_copy(NU - 1, 1).wait()

    return k(x3, order)


def kernel(X, shuffled_idx):
    rand_idx = jax.random.randint(jax.random.key(1), (1,), 0, NUM_PERM - 1)[0]
    order = lax.dynamic_index_in_dim(
        shuffled_idx, rand_idx, axis=0, keepdims=False
    ).astype(jnp.int32)
    x3 = X.reshape(B * C, E, P)
    out = _shuffle_gather(x3, order)
    return out.reshape(B, C, E, P)


# speedup vs baseline: 15.2914x; 1.0077x over previous
import functools

import jax
import jax.numpy as jnp
from jax import lax
from jax.experimental import pallas as pl
from jax.experimental.pallas import tpu as pltpu
from jax.experimental.pallas import tpu_sc as plsc

B, C, E, P = 32, 16, 128, 512
NUM_PERM = 1000

NC, NS, L = 2, 16, 16
NW = NC * NS
PAIRS_PER_W = (B * C) // NW
CE = 32
NCH = E // CE
NU = PAIRS_PER_W * NCH


def _shuffle_gather(x3, order):
    mesh = plsc.VectorSubcoreMesh(core_axis_name="c", subcore_axis_name="s")

    @functools.partial(
        pl.kernel,
        out_type=jax.ShapeDtypeStruct((B * C, E, P), jnp.float32),
        mesh=mesh,
        compiler_params=pltpu.CompilerParams(needs_layout_passes=False),
        scratch_types=[
            pltpu.VMEM((P,), jnp.int32),
            pltpu.VMEM((CE, P), jnp.float32),
            pltpu.VMEM((CE, P), jnp.float32),
            pltpu.VMEM((CE, P), jnp.float32),
            pltpu.VMEM((CE, P), jnp.float32),
            pltpu.SemaphoreType.DMA,
            pltpu.SemaphoreType.DMA,
            pltpu.SemaphoreType.DMA,
            pltpu.SemaphoreType.DMA,
        ],
    )
    def k(x_hbm, idx_hbm, out_hbm, idxbuf, in0, in1, out0, out1,
          isem0, isem1, osem0, osem1):
        cid = lax.axis_index("c")
        sid = lax.axis_index("s")
        w = sid * NC + cid
        c = w // 2
        bh = w % 2
        inb = (in0, in1)
        outb = (out0, out1)
        isem = (isem0, isem1)
        osem = (osem0, osem1)

        pltpu.sync_copy(idx_hbm.at[c], idxbuf)

        def unit_slot(u):
            kk = u // NCH
            ch = u % NCH
            b = bh * (B // 2) + kk
            return b * C + c, ch * CE

        def in_copy(u, par):
            pair, e0 = unit_slot(u)
            return pltpu.make_async_copy(
                x_hbm.at[pair, pl.ds(e0, CE), :], inb[par], isem[par])

        def out_copy(u, par):
            pair, e0 = unit_slot(u)
            return pltpu.make_async_copy(
                outb[par], out_hbm.at[pair, pl.ds(e0, CE), :], osem[par])

        def compute(inbuf, outbuf):
            @plsc.parallel_loop(0, P // L, unroll=8)
            def j_body(j):
                iv = idxbuf[pl.ds(j * L, L)]

                @plsc.parallel_loop(0, CE, unroll=8)
                def e_body(e):
                    ev = jnp.full((L,), e, dtype=jnp.int32)
                    outbuf[e, pl.ds(j * L, L)] = plsc.load_gather(
                        inbuf, [ev, iv])

        in_copy(0, 0).start()

        def outer(i, _):
            for par in range(2):
                u = i * 2 + par

                @pl.when(u + 1 < NU)
                def _start_next():
                    in_copy(u + 1, 1 - par).start()

                in_copy(u, par).wait()

                @pl.when(u >= 2)
                def _drain_prev():
                    out_copy(u - 2, par).wait()

                compute(inb[par], outb[par])
                out_copy(u, par).start()
            return 0

        lax.fori_loop(0, NU // 2, outer, 0)
        out_copy(NU - 2, 0).wait()
        out_copy(NU - 1, 1).wait()

    return k(x3, order)


def kernel(X, shuffled_idx):
    rand_idx = jax.random.randint(jax.random.key(1), (1,), 0, NUM_PERM - 1)[0]
    order = lax.dynamic_index_in_dim(
        shuffled_idx, rand_idx, axis=0, keepdims=False
    ).astype(jnp.int32)
    x3 = X.reshape(B * C, E, P)
    out = _shuffle_gather(x3, order)
    return out.reshape(B, C, E, P)
